# Initial kernel scaffold; baseline (speedup 1.0000x reference)
import jax
import jax.numpy as jnp
from jax.experimental import pallas as pl


def kernel(x, y, z, g_src, g_dst, lg_src, lg_dst, params):
    raise NotImplementedError("write your pallas kernel here")


# speedup vs baseline: 1.0422x; 1.0422x over previous
import functools

import jax
import jax.numpy as jnp
from jax import lax
from jax.experimental import pallas as pl
from jax.experimental.pallas import tpu as pltpu


_BLK = 2000


def _mm_body(x_ref, w_ref, b_ref, o_ref):
    o_ref[...] = (
        jnp.dot(x_ref[...], w_ref[...], preferred_element_type=jnp.float32)
        + b_ref[...]
    )


def _fused_matmul(x, w, b):
    r, d = x.shape
    k = w.shape[1]
    grid = (r // _BLK,)
    return pl.pallas_call(
        _mm_body,
        grid=grid,
        in_specs=[
            pl.BlockSpec((_BLK, d), lambda i: (i, 0)),
            pl.BlockSpec((d, k), lambda i: (0, 0)),
            pl.BlockSpec((1, k), lambda i: (0, 0)),
        ],
        out_specs=pl.BlockSpec((_BLK, k), lambda i: (i, 0)),
        out_shape=jax.ShapeDtypeStruct((r, k), jnp.float32),
    )(x, w, b)


def _ln_silu(v, gamma, beta):
    mu = jnp.mean(v, axis=-1, keepdims=True)
    var = jnp.mean((v - mu) ** 2, axis=-1, keepdims=True)
    ln = (v - mu) * lax.rsqrt(var + 1e-5) * gamma + beta
    return ln * jax.nn.sigmoid(ln)


def _edge_stage_body(gs_ref, gd_ref, ef_ref, gb_ref, w_ref, b_ref,
                     gam_ref, bet_ref, sig_ref, p_ref, eout_ref):
    ef = ef_ref[...]
    m = (
        gs_ref[...]
        + gd_ref[...]
        + jnp.dot(ef, w_ref[...], preferred_element_type=jnp.float32)
        + b_ref[...]
    )
    sig = jax.nn.sigmoid(m)
    sig_ref[...] = sig
    p_ref[...] = gb_ref[...] * sig
    eout_ref[...] = ef + _ln_silu(m, gam_ref[...], bet_ref[...])


def _edge_stage(gs, gd, ef, gb, w_eg, b_eg, gamma, beta):
    r, d = ef.shape
    grid = (r // _BLK,)
    blk = lambda i: (i, 0)
    full = lambda i: (0, 0)
    return pl.pallas_call(
        _edge_stage_body,
        grid=grid,
        in_specs=[
            pl.BlockSpec((_BLK, d), blk),
            pl.BlockSpec((_BLK, d), blk),
            pl.BlockSpec((_BLK, d), blk),
            pl.BlockSpec((_BLK, d), blk),
            pl.BlockSpec((d, d), full),
            pl.BlockSpec((1, d), full),
            pl.BlockSpec((1, d), full),
            pl.BlockSpec((1, d), full),
        ],
        out_specs=[pl.BlockSpec((_BLK, d), blk)] * 3,
        out_shape=[jax.ShapeDtypeStruct((r, d), jnp.float32)] * 3,
    )(gs, gd, ef, gb, w_eg, b_eg, gamma, beta)


def _node_stage_body(u_ref, num_ref, den_ref, res_ref, gam_ref, bet_ref,
                     o_ref):
    v = u_ref[...] + num_ref[...] / (den_ref[...] + 1e-6)
    v = _ln_silu(v, gam_ref[...], bet_ref[...])
    o_ref[...] = res_ref[...] + v


def _node_stage(u, num, den, res, gamma, beta):
    r, d = u.shape
    grid = (r // _BLK,)
    blk = lambda i: (i, 0)
    full = lambda i: (0, 0)
    return pl.pallas_call(
        _node_stage_body,
        grid=grid,
        in_specs=[
            pl.BlockSpec((_BLK, d), blk),
            pl.BlockSpec((_BLK, d), blk),
            pl.BlockSpec((_BLK, d), blk),
            pl.BlockSpec((_BLK, d), blk),
            pl.BlockSpec((1, d), full),
            pl.BlockSpec((1, d), full),
        ],
        out_specs=pl.BlockSpec((_BLK, d), blk),
        out_shape=jax.ShapeDtypeStruct((r, d), jnp.float32),
    )(u, num, den, res, gamma, beta)


def _gather_rows(tables, idxs):
    return [t[i] for t, i in zip(tables, idxs)]


def _segment_sums(sig, p, dst, n):
    num = jax.ops.segment_sum(p, dst, num_segments=n)
    den = jax.ops.segment_sum(sig, dst, num_segments=n)
    return num, den


def _cat_w(p, names):
    return jnp.concatenate([p[n]["w"].T for n in names], axis=1)


def _cat_b(p, names):
    return jnp.concatenate([p[n]["b"] for n in names])[None, :]


_NAMES = ["src_gate", "dst_gate", "dst_update", "src_update"]


def _egc_layer(p, src, dst, n_nodes, node_feats, edge_feats):
    d = node_feats.shape[1]
    cat = _fused_matmul(node_feats, _cat_w(p, _NAMES), _cat_b(p, _NAMES))
    sx, dx, bx, ux = (cat[:, i * d:(i + 1) * d] for i in range(4))
    gs, gd, gb = _gather_rows([sx, dx, bx], [src, dst, src])
    sig, pprod, e_out = _edge_stage(
        gs, gd, edge_feats, gb,
        p["edge_gate"]["w"].T, p["edge_gate"]["b"][None, :],
        p["norm_edges"]["gamma"][None, :], p["norm_edges"]["beta"][None, :],
    )
    num, den = _segment_sums(sig, pprod, dst, n_nodes)
    n_out = _node_stage(
        ux, num, den, node_feats,
        p["norm_nodes"]["gamma"][None, :], p["norm_nodes"]["beta"][None, :],
    )
    return n_out, e_out


def kernel(x, y, z, g_src, g_dst, lg_src, lg_dst, params):
    g_src = g_src.astype(jnp.int32)
    g_dst = g_dst.astype(jnp.int32)
    lg_src = lg_src.astype(jnp.int32)
    lg_dst = lg_dst.astype(jnp.int32)

    x_out, m = _egc_layer(params["node_update"], g_src, g_dst, x.shape[0], x, y)
    y_out, z_out = _egc_layer(params["edge_update"], lg_src, lg_dst,
                              m.shape[0], m, z)
    return (x_out, y_out, z_out)
